# baseline (speedup 1.0000x reference)
import functools

import jax
import jax.numpy as jnp
from jax import lax
from jax.experimental import pallas as pl
from jax.experimental.pallas import tpu as pltpu
from jax.experimental.pallas import tpu_sc as plsc

_B = 16384
_D = 16
_V = 1000000
_GROUP = 8
_NG = _V // _GROUP
_NC = 2
_NS = 16
_NW = _NC * _NS
_CHUNK = 128
_ROWS_PER_W = _B // _NW
_NCH = _ROWS_PER_W // _CHUNK


def _sc_gather_body(uidx_hbm, iidx_hbm, w_hbm, h_hbm, h1_hbm,
                    u_out, v_out, v1_out,
                    uidx_v, iidx_v, bufs0, bufs1, sem):
    wid = lax.axis_index("s") * _NC + lax.axis_index("c")
    base = wid * _ROWS_PER_W
    pltpu.sync_copy(uidx_hbm.at[wid], uidx_v)
    pltpu.sync_copy(iidx_hbm.at[wid], iidx_v)
    bufs = (bufs0, bufs1)
    outs = (u_out, v_out, v1_out)

    def fire(j):
        u_b, v_b, v1_b = bufs[j % 2]
        return [
            pltpu.async_copy(w_hbm.at[uidx_v.at[j]], u_b, sem),
            pltpu.async_copy(h_hbm.at[iidx_v.at[j]], v_b, sem),
            pltpu.async_copy(h1_hbm.at[iidx_v.at[j]], v1_b, sem),
        ]

    def drain(j, copies):
        for c in copies:
            c.wait()
        sl = pl.ds(base + j * _CHUNK, _CHUNK)
        for buf, out in zip(bufs[j % 2], outs):
            pltpu.sync_copy(buf, out.at[sl])

    inflight = fire(0)
    for j in range(1, _NCH):
        nxt = fire(j)
        drain(j - 1, inflight)
        inflight = nxt
    drain(_NCH - 1, inflight)


@functools.lru_cache(maxsize=None)
def _sc_gather():
    row_buf = lambda: [pltpu.VMEM((_CHUNK, 128), jnp.float32) for _ in range(3)]
    return functools.partial(
        pl.kernel,
        out_type=[jax.ShapeDtypeStruct((_B, 128), jnp.float32)] * 3,
        mesh=plsc.VectorSubcoreMesh(core_axis_name="c", subcore_axis_name="s"),
        scratch_types=[
            pltpu.VMEM((_NCH, _CHUNK), jnp.int32),
            pltpu.VMEM((_NCH, _CHUNK), jnp.int32),
            row_buf(),
            row_buf(),
            pltpu.SemaphoreType.DMA,
        ],
    )(_sc_gather_body)


def _select16(raw, sub, fold):
    lane_grp = lax.broadcasted_iota(jnp.int32, raw.shape, 1) // _D
    mask = (lane_grp == sub).astype(jnp.float32)
    return jnp.dot(raw * mask, fold, preferred_element_type=jnp.float32)


def _mlp_body(drug_ref, u_ref, v_ref, v1_ref, su_ref, si_ref,
              ww1_ref, wb1_ref, ww2_ref, wb2_ref, dw1_ref, db1_ref,
              dw2_ref, g_ref, out_ref):
    fold = (lax.broadcasted_iota(jnp.int32, (128, _D), 0) % _D ==
            lax.broadcasted_iota(jnp.int32, (128, _D), 1)).astype(jnp.float32)
    su = su_ref[...]
    si = si_ref[...]
    u = _select16(u_ref[...], su, fold)
    v = _select16(v_ref[...], si, fold)
    v1 = _select16(v1_ref[...], si, fold)
    drug = drug_ref[...]
    wh = jnp.maximum(
        jnp.dot(drug, ww1_ref[...], preferred_element_type=jnp.float32)
        + wb1_ref[...], 0.0)
    wide = (jnp.dot(wh, ww2_ref[...], preferred_element_type=jnp.float32)
            + wb2_ref[...]) * v1
    wide_t = jnp.sum(wide, axis=1, keepdims=True)
    z = jnp.concatenate([u, v], axis=1)
    h = jax.nn.sigmoid(
        jnp.dot(z, dw1_ref[...], preferred_element_type=jnp.float32)
        + db1_ref[...])
    dnn = jnp.dot(h, dw2_ref[...], preferred_element_type=jnp.float32)
    gw = g_ref[0, 0]
    gb = g_ref[0, 1]
    out_ref[...] = jax.nn.sigmoid(wide_t * gw + gb + dnn)[:, 0]


def _mlp_call(blk):
    grid = _B // blk
    full = lambda shape: pl.BlockSpec(shape, lambda i: (0, 0))
    return pl.pallas_call(
        _mlp_body,
        grid=(grid,),
        in_specs=[
            pl.BlockSpec((blk, 256), lambda i: (i, 0)),
            pl.BlockSpec((blk, 128), lambda i: (i, 0)),
            pl.BlockSpec((blk, 128), lambda i: (i, 0)),
            pl.BlockSpec((blk, 128), lambda i: (i, 0)),
            pl.BlockSpec((blk, 1), lambda i: (i, 0)),
            pl.BlockSpec((blk, 1), lambda i: (i, 0)),
            full((256, 64)),
            full((1, 64)),
            full((64, _D)),
            full((1, _D)),
            full((2 * _D, _D)),
            full((1, _D)),
            full((_D, 1)),
            full((1, 2)),
        ],
        out_specs=pl.BlockSpec((blk,), lambda i: (i,)),
        out_shape=jax.ShapeDtypeStruct((_B,), jnp.float32),
    )


def kernel(x, drug_features_x, W, H, H1, wide_w1, wide_b1, wide_w2, wide_b2,
           deep_w1, deep_b1, deep_w2, g_w, g_b):
    xi = x.astype(jnp.int32)
    ugrp = (xi[:, 0] // _GROUP).reshape(_NW, _NCH, _CHUNK)
    igrp = (xi[:, 1] // _GROUP).reshape(_NW, _NCH, _CHUNK)
    usub = (xi[:, 0] % _GROUP).reshape(_B, 1)
    isub = (xi[:, 1] % _GROUP).reshape(_B, 1)
    w128 = W[:_V].reshape(_NG, 128)
    h128 = H[:_V].reshape(_NG, 128)
    h1128 = H1[:_V].reshape(_NG, 128)
    u_raw, v_raw, v1_raw = _sc_gather()(ugrp, igrp, w128, h128, h1128)
    g = jnp.concatenate([g_w.reshape(1, 1), g_b.reshape(1, 1)], axis=1)
    out = _mlp_call(2048)(
        drug_features_x, u_raw, v_raw, v1_raw, usub, isub,
        wide_w1, wide_b1.reshape(1, 64), wide_w2, wide_b2.reshape(1, _D),
        deep_w1, deep_b1.reshape(1, _D), deep_w2, g)
    return out
